# initial kernel scaffold (unmeasured)
import jax
import jax.numpy as jnp
from jax import lax
from jax.experimental import pallas as pl
from jax.experimental.pallas import tpu as pltpu

N_DEV = 8


def kernel(x, w_mat, scale_x, scale_w):
    m, k_per = x.shape
    _, n = w_mat.shape
    m_per = m // N_DEV

    def body(x_ref, w_ref, sx_ref, sw_ref, out_ref, comm_ref, send_sems, recv_sems):
        my = lax.axis_index("i")
        left = lax.rem(my + N_DEV - 1, N_DEV)
        right = lax.rem(my + 1, N_DEV)

        barrier = pltpu.get_barrier_semaphore()
        for nbr in (left, right):
            pl.semaphore_signal(
                barrier, inc=1, device_id=(nbr,),
                device_id_type=pl.DeviceIdType.MESH,
            )
        pl.semaphore_wait(barrier, 2)

        w_bf = w_ref[:, :].astype(jnp.bfloat16)

        def partial(c):
            xb = x_ref[pl.ds(c * m_per, m_per), :].astype(jnp.bfloat16)
            return jnp.dot(xb, w_bf, preferred_element_type=jnp.float32)

        comm_ref[N_DEV - 1] = partial(lax.rem(my + N_DEV - 1, N_DEV))

        for s in range(N_DEV - 1):
            src_slot = (N_DEV - 1) if s == 0 else s - 1
            rdma = pltpu.make_async_remote_copy(
                src_ref=comm_ref.at[src_slot],
                dst_ref=comm_ref.at[s],
                send_sem=send_sems.at[s],
                recv_sem=recv_sems.at[s],
                device_id=(right,),
                device_id_type=pl.DeviceIdType.MESH,
            )
            rdma.start()
            c = lax.rem(my + 2 * N_DEV - s - 2, N_DEV)
            p = partial(c)
            rdma.wait()
            if s < N_DEV - 2:
                comm_ref[s] = comm_ref[s] + p
            else:
                acc = comm_ref[s] + p
                y = acc * (sx_ref[0] * sw_ref[0])
                out_ref[:, :] = y * jax.nn.sigmoid(y)

    return pl.pallas_call(
        body,
        out_shape=jax.ShapeDtypeStruct((m_per, n), jnp.float32),
        in_specs=[
            pl.BlockSpec(memory_space=pltpu.VMEM),
            pl.BlockSpec(memory_space=pltpu.VMEM),
            pl.BlockSpec(memory_space=pltpu.SMEM),
            pl.BlockSpec(memory_space=pltpu.SMEM),
        ],
        out_specs=pl.BlockSpec(memory_space=pltpu.VMEM),
        scratch_shapes=[
            pltpu.VMEM((N_DEV, m_per, n), jnp.float32),
            pltpu.SemaphoreType.DMA((N_DEV - 1,)),
            pltpu.SemaphoreType.DMA((N_DEV - 1,)),
        ],
        compiler_params=pltpu.CompilerParams(collective_id=0),
    )(x, w_mat, scale_x, scale_w)


# baseline (device time: 347101 ns/iter reference)
import jax
import jax.numpy as jnp
from jax import lax
from jax.experimental import pallas as pl
from jax.experimental.pallas import tpu as pltpu

N_DEV = 8


def kernel(x, w_mat, scale_x, scale_w):
    m, k_per = x.shape
    _, n = w_mat.shape
    m_per = m // N_DEV

    def body(x_ref, w_ref, sx_ref, sw_ref, out_ref, comm_ref, send_sems, recv_sems):
        my = lax.axis_index("i")
        left = lax.rem(my + N_DEV - 1, N_DEV)
        right = lax.rem(my + 1, N_DEV)

        barrier = pltpu.get_barrier_semaphore()
        for nbr in (left, right):
            pl.semaphore_signal(
                barrier, inc=1, device_id=(nbr,),
                device_id_type=pl.DeviceIdType.MESH,
            )
        pl.semaphore_wait(barrier, 2)

        w_bf = w_ref[:, :].astype(jnp.bfloat16)

        def partial(c):
            xb = x_ref[pl.ds(c * m_per, m_per), :].astype(jnp.bfloat16)
            return jnp.dot(xb, w_bf, preferred_element_type=jnp.float32)

        comm_ref[N_DEV - 1] = partial(lax.rem(my + N_DEV - 1, N_DEV))

        for s in range(N_DEV - 1):
            src_slot = (N_DEV - 1) if s == 0 else s - 1
            rdma = pltpu.make_async_remote_copy(
                src_ref=comm_ref.at[src_slot],
                dst_ref=comm_ref.at[s],
                send_sem=send_sems.at[s],
                recv_sem=recv_sems.at[s],
                device_id=(right,),
                device_id_type=pl.DeviceIdType.MESH,
            )
            rdma.start()
            c = lax.rem(my + 2 * N_DEV - s - 2, N_DEV)
            p = partial(c)
            rdma.wait()
            if s < N_DEV - 2:
                comm_ref[s] = comm_ref[s] + p
            else:
                acc = comm_ref[s] + p
                y = acc * (sx_ref[0] * sw_ref[0])
                out_ref[:, :] = y * jax.nn.sigmoid(y)

    return pl.pallas_call(
        body,
        out_shape=jax.ShapeDtypeStruct((m_per, n), jnp.float32),
        in_specs=[
            pl.BlockSpec(memory_space=pltpu.VMEM),
            pl.BlockSpec(memory_space=pltpu.VMEM),
            pl.BlockSpec(memory_space=pltpu.SMEM),
            pl.BlockSpec(memory_space=pltpu.SMEM),
        ],
        out_specs=pl.BlockSpec(memory_space=pltpu.VMEM),
        scratch_shapes=[
            pltpu.VMEM((N_DEV, m_per, n), jnp.float32),
            pltpu.SemaphoreType.DMA((N_DEV - 1,)),
            pltpu.SemaphoreType.DMA((N_DEV - 1,)),
        ],
        compiler_params=pltpu.CompilerParams(
            collective_id=0,
            vmem_limit_bytes=100 * 1024 * 1024,
        ),
    )(x, w_mat, scale_x, scale_w)


# device time: 191910 ns/iter; 1.8087x vs baseline; 1.8087x over previous
import jax
import jax.numpy as jnp
from jax import lax
from jax.experimental import pallas as pl
from jax.experimental.pallas import tpu as pltpu

N_DEV = 8


def kernel(x, w_mat, scale_x, scale_w):
    m, k_per = x.shape
    _, n = w_mat.shape
    m_per = m // N_DEV

    def body(x_ref, w_ref, sx_ref, sw_ref, out_ref, comm_ref, send_sems, recv_sems):
        my = lax.axis_index("i")
        left = lax.rem(my + N_DEV - 1, N_DEV)
        right = lax.rem(my + 1, N_DEV)

        barrier = pltpu.get_barrier_semaphore()
        for nbr in (left, right):
            pl.semaphore_signal(
                barrier, inc=1, device_id=(nbr,),
                device_id_type=pl.DeviceIdType.MESH,
            )
        pl.semaphore_wait(barrier, 2)

        w_bf = w_ref[:, :].astype(jnp.bfloat16)

        def partial(c):
            xb = x_ref[pl.ds(c * m_per, m_per), :].astype(jnp.bfloat16)
            return jnp.dot(xb, w_bf, preferred_element_type=jnp.float32)

        comm_ref[N_DEV - 1] = partial(lax.rem(my + N_DEV - 1, N_DEV)).astype(jnp.bfloat16)

        for s in range(N_DEV - 1):
            src_slot = (N_DEV - 1) if s == 0 else s - 1
            rdma = pltpu.make_async_remote_copy(
                src_ref=comm_ref.at[src_slot],
                dst_ref=comm_ref.at[s],
                send_sem=send_sems.at[s],
                recv_sem=recv_sems.at[s],
                device_id=(right,),
                device_id_type=pl.DeviceIdType.MESH,
            )
            rdma.start()
            c = lax.rem(my + 2 * N_DEV - s - 2, N_DEV)
            p = partial(c)
            rdma.wait()
            if s < N_DEV - 2:
                comm_ref[s] = (comm_ref[s].astype(jnp.float32) + p).astype(jnp.bfloat16)
            else:
                acc = comm_ref[s].astype(jnp.float32) + p
                y = acc * (sx_ref[0] * sw_ref[0])
                out_ref[:, :] = y * jax.nn.sigmoid(y)

    return pl.pallas_call(
        body,
        out_shape=jax.ShapeDtypeStruct((m_per, n), jnp.float32),
        in_specs=[
            pl.BlockSpec(memory_space=pltpu.VMEM),
            pl.BlockSpec(memory_space=pltpu.VMEM),
            pl.BlockSpec(memory_space=pltpu.SMEM),
            pl.BlockSpec(memory_space=pltpu.SMEM),
        ],
        out_specs=pl.BlockSpec(memory_space=pltpu.VMEM),
        scratch_shapes=[
            pltpu.VMEM((N_DEV, m_per, n), jnp.bfloat16),
            pltpu.SemaphoreType.DMA((N_DEV - 1,)),
            pltpu.SemaphoreType.DMA((N_DEV - 1,)),
        ],
        compiler_params=pltpu.CompilerParams(
            collective_id=0,
            vmem_limit_bytes=100 * 1024 * 1024,
        ),
    )(x, w_mat, scale_x, scale_w)


# device time: 116059 ns/iter; 2.9907x vs baseline; 1.6536x over previous
import jax
import jax.numpy as jnp
from jax import lax
from jax.experimental import pallas as pl
from jax.experimental.pallas import tpu as pltpu

N_DEV = 8


def kernel(x, w_mat, scale_x, scale_w):
    m, k_per = x.shape
    _, n = w_mat.shape
    m_per = m // N_DEV
    n_half = n // 2

    def body(x_ref, w_ref, sx_ref, sw_ref, out_ref,
             commR, commL, sendR, recvR, sendL, recvL):
        my = lax.axis_index("i")
        left = lax.rem(my + N_DEV - 1, N_DEV)
        right = lax.rem(my + 1, N_DEV)

        barrier = pltpu.get_barrier_semaphore()
        for nbr in (left, right):
            pl.semaphore_signal(
                barrier, inc=1, device_id=(nbr,),
                device_id_type=pl.DeviceIdType.MESH,
            )
        pl.semaphore_wait(barrier, 2)

        w_lo = w_ref[:, :n_half].astype(jnp.bfloat16)
        w_hi = w_ref[:, n_half:].astype(jnp.bfloat16)

        def partial(c, w_half):
            xb = x_ref[pl.ds(c * m_per, m_per), :].astype(jnp.bfloat16)
            return jnp.dot(xb, w_half, preferred_element_type=jnp.float32)

        commR[N_DEV - 1] = partial(lax.rem(my + N_DEV - 1, N_DEV), w_lo).astype(jnp.bfloat16)
        commL[N_DEV - 1] = partial(lax.rem(my + 1, N_DEV), w_hi).astype(jnp.bfloat16)

        for s in range(N_DEV - 1):
            src = (N_DEV - 1) if s == 0 else s - 1
            rdmaR = pltpu.make_async_remote_copy(
                src_ref=commR.at[src], dst_ref=commR.at[s],
                send_sem=sendR.at[s], recv_sem=recvR.at[s],
                device_id=(right,), device_id_type=pl.DeviceIdType.MESH,
            )
            rdmaL = pltpu.make_async_remote_copy(
                src_ref=commL.at[src], dst_ref=commL.at[s],
                send_sem=sendL.at[s], recv_sem=recvL.at[s],
                device_id=(left,), device_id_type=pl.DeviceIdType.MESH,
            )
            rdmaR.start()
            rdmaL.start()
            cR = lax.rem(my + 2 * N_DEV - s - 2, N_DEV)
            cL = lax.rem(my + s + 2, N_DEV)
            pR = partial(cR, w_lo)
            pL = partial(cL, w_hi)
            rdmaR.wait()
            rdmaL.wait()
            if s < N_DEV - 2:
                commR[s] = (commR[s].astype(jnp.float32) + pR).astype(jnp.bfloat16)
                commL[s] = (commL[s].astype(jnp.float32) + pL).astype(jnp.bfloat16)
            else:
                scale = sx_ref[0] * sw_ref[0]
                yR = (commR[s].astype(jnp.float32) + pR) * scale
                yL = (commL[s].astype(jnp.float32) + pL) * scale
                out_ref[:, :n_half] = yR * jax.nn.sigmoid(yR)
                out_ref[:, n_half:] = yL * jax.nn.sigmoid(yL)

    return pl.pallas_call(
        body,
        out_shape=jax.ShapeDtypeStruct((m_per, n), jnp.float32),
        in_specs=[
            pl.BlockSpec(memory_space=pltpu.VMEM),
            pl.BlockSpec(memory_space=pltpu.VMEM),
            pl.BlockSpec(memory_space=pltpu.SMEM),
            pl.BlockSpec(memory_space=pltpu.SMEM),
        ],
        out_specs=pl.BlockSpec(memory_space=pltpu.VMEM),
        scratch_shapes=[
            pltpu.VMEM((N_DEV, m_per, n_half), jnp.bfloat16),
            pltpu.VMEM((N_DEV, m_per, n_half), jnp.bfloat16),
            pltpu.SemaphoreType.DMA((N_DEV - 1,)),
            pltpu.SemaphoreType.DMA((N_DEV - 1,)),
            pltpu.SemaphoreType.DMA((N_DEV - 1,)),
            pltpu.SemaphoreType.DMA((N_DEV - 1,)),
        ],
        compiler_params=pltpu.CompilerParams(
            collective_id=0,
            vmem_limit_bytes=100 * 1024 * 1024,
        ),
    )(x, w_mat, scale_x, scale_w)


# device time: 86625 ns/iter; 4.0069x vs baseline; 1.3398x over previous
import jax
import jax.numpy as jnp
from jax import lax
from jax.experimental import pallas as pl
from jax.experimental.pallas import tpu as pltpu

N_DEV = 8

GROUPS = (
    ((2, 1, 0), 640),
    ((1, 0, 2), 640),
    ((0, 2, 1), 768),
)


def kernel(x, w_mat, scale_x, scale_w):
    m, k_per = x.shape
    _, n = w_mat.shape
    m_per = m // N_DEV
    assert sum(w for _, w in GROUPS) == n

    def body(x_ref, w_ref, sx_ref, sw_ref, out_ref, *scratch):
        sc = [scratch[9 * g:9 * (g + 1)] for g in range(len(GROUPS))]

        my = lax.axis_index("i")
        p4 = lax.rem(my, 4)
        mbit = (
            lax.rem(lax.rem(p4, 2) + p4 // 2, 2),
            p4 // 2,
            my // 4,
        )

        def pos_from_bits(bx, by, bz):
            return 4 * bz + bx + by * (3 - 2 * bx)

        def partner(axis):
            b = list(mbit)
            b[axis] = 1 - b[axis]
            return pos_from_bits(*b)

        def chunk_c(bits_by_axis):
            return pos_from_bits(bits_by_axis[0], bits_by_axis[1], bits_by_axis[2])

        barrier = pltpu.get_barrier_semaphore()
        for a in range(3):
            pl.semaphore_signal(
                barrier, inc=1, device_id=(partner(a),),
                device_id_type=pl.DeviceIdType.MESH,
            )
        pl.semaphore_wait(barrier, 3)

        col0s = []
        w_bfs = []
        c0 = 0
        for (_, w) in GROUPS:
            col0s.append(c0)
            w_bfs.append(w_ref[:, c0:c0 + w].astype(jnp.bfloat16))
            c0 += w

        def partial(c, w_bf):
            xb = x_ref[pl.ds(c * m_per, m_per), :].astype(jnp.bfloat16)
            return jnp.dot(xb, w_bf, preferred_element_type=jnp.float32)

        def mk(src, dst, sems, recvs, stage, axis):
            return pltpu.make_async_remote_copy(
                src_ref=src, dst_ref=dst,
                send_sem=sems.at[stage], recv_sem=recvs.at[stage],
                device_id=(partner(axis),),
                device_id_type=pl.DeviceIdType.MESH,
            )

        def slot_bits(axes, b1, j):
            bits = [None, None, None]
            bits[axes[0]] = b1
            bits[axes[1]] = j // 2
            bits[axes[2]] = j % 2
            return bits

        rd1 = []
        for g, (axes, w) in enumerate(GROUPS):
            acc, sb1, rb1, sb2, rb2, sb3, rb3, ss, rs = sc[g]
            m1 = mbit[axes[0]]
            for j in range(4):
                c = chunk_c(slot_bits(axes, 1 - m1, j))
                sb1[j] = partial(c, w_bfs[g]).astype(jnp.bfloat16)
            r = mk(sb1, rb1, ss, rs, 0, axes[0])
            r.start()
            rd1.append(r)
        for g, (axes, w) in enumerate(GROUPS):
            acc = sc[g][0]
            m1 = mbit[axes[0]]
            for j in range(4):
                c = chunk_c(slot_bits(axes, m1, j))
                acc[j] = partial(c, w_bfs[g])
        for g, (axes, w) in enumerate(GROUPS):
            acc, sb1, rb1 = sc[g][0], sc[g][1], sc[g][2]
            rd1[g].wait()
            acc[:, :, :] = acc[:, :, :] + rb1[:, :, :].astype(jnp.float32)

        rd2 = []
        for g, (axes, w) in enumerate(GROUPS):
            acc, sb2, rb2, ss, rs = sc[g][0], sc[g][3], sc[g][4], sc[g][7], sc[g][8]
            m2 = mbit[axes[1]]
            sb2[:, :, :] = acc[pl.ds(2 * (1 - m2), 2)].astype(jnp.bfloat16)
            r = mk(sb2, rb2, ss, rs, 1, axes[1])
            r.start()
            rd2.append(r)
        for g, (axes, w) in enumerate(GROUPS):
            acc, rb2 = sc[g][0], sc[g][4]
            m2 = mbit[axes[1]]
            rd2[g].wait()
            keep = pl.ds(2 * m2, 2)
            acc[keep] = acc[keep] + rb2[:, :, :].astype(jnp.float32)

        rd3 = []
        for g, (axes, w) in enumerate(GROUPS):
            acc, sb3, rb3, ss, rs = sc[g][0], sc[g][5], sc[g][6], sc[g][7], sc[g][8]
            m2, m3 = mbit[axes[1]], mbit[axes[2]]
            sb3[:, :, :] = acc[pl.ds(2 * m2 + (1 - m3), 1)].astype(jnp.bfloat16)
            r = mk(sb3, rb3, ss, rs, 2, axes[2])
            r.start()
            rd3.append(r)
        scale = sx_ref[0] * sw_ref[0]
        for g, (axes, w) in enumerate(GROUPS):
            acc, rb3 = sc[g][0], sc[g][6]
            m2, m3 = mbit[axes[1]], mbit[axes[2]]
            rd3[g].wait()
            final = acc[pl.ds(2 * m2 + m3, 1)][0] + rb3[0].astype(jnp.float32)
            y = final * scale
            out_ref[:, col0s[g]:col0s[g] + w] = y * jax.nn.sigmoid(y)

    scratch = []
    for (_, w) in GROUPS:
        scratch += [
            pltpu.VMEM((4, m_per, w), jnp.float32),
            pltpu.VMEM((4, m_per, w), jnp.bfloat16),
            pltpu.VMEM((4, m_per, w), jnp.bfloat16),
            pltpu.VMEM((2, m_per, w), jnp.bfloat16),
            pltpu.VMEM((2, m_per, w), jnp.bfloat16),
            pltpu.VMEM((1, m_per, w), jnp.bfloat16),
            pltpu.VMEM((1, m_per, w), jnp.bfloat16),
            pltpu.SemaphoreType.DMA((3,)),
            pltpu.SemaphoreType.DMA((3,)),
        ]

    return pl.pallas_call(
        body,
        out_shape=jax.ShapeDtypeStruct((m_per, n), jnp.float32),
        in_specs=[
            pl.BlockSpec(memory_space=pltpu.VMEM),
            pl.BlockSpec(memory_space=pltpu.VMEM),
            pl.BlockSpec(memory_space=pltpu.SMEM),
            pl.BlockSpec(memory_space=pltpu.SMEM),
        ],
        out_specs=pl.BlockSpec(memory_space=pltpu.VMEM),
        scratch_shapes=scratch,
        compiler_params=pltpu.CompilerParams(
            collective_id=0,
            vmem_limit_bytes=110 * 1024 * 1024,
        ),
    )(x, w_mat, scale_x, scale_w)
